# Initial kernel scaffold; baseline (speedup 1.0000x reference)
import jax
import jax.numpy as jnp
from jax.experimental import pallas as pl


def kernel(x, router_w, w1, w2):
    raise NotImplementedError("write your pallas kernel here")


# speedup vs baseline: 2.4073x; 2.4073x over previous
import functools

import jax
import jax.numpy as jnp
from jax import lax
from jax.experimental import pallas as pl
from jax.experimental.pallas import tpu as pltpu

E = 8
TOPK = 2
D = 1024
FF = 4096
T = 2048

TB = 256
NBLK_MAX = (T * TOPK) // TB + E
R = NBLK_MAX * TB
FSPLIT = 2
FFC = FF // FSPLIT
JMAX = T // TB


def _router_body(x_ref, rw_ref, wts_ref, idx_ref):
    x = x_ref[...]
    rw = rw_ref[...]
    logits = lax.dot_general(x, rw, (((1,), (1,)), ((), ())),
                             preferred_element_type=jnp.float32)
    m = jnp.max(logits, axis=1, keepdims=True)
    ex = jnp.exp(logits - m)
    probs = ex / jnp.sum(ex, axis=1, keepdims=True)
    iota = lax.broadcasted_iota(jnp.int32, (T, E), 1)
    m1 = jnp.max(probs, axis=1, keepdims=True)
    i1 = jnp.min(jnp.where(probs == m1, iota, E), axis=1, keepdims=True)
    probs2 = jnp.where(iota == i1, -jnp.inf, probs)
    m2 = jnp.max(probs2, axis=1, keepdims=True)
    i2 = jnp.min(jnp.where(probs2 == m2, iota, E), axis=1, keepdims=True)
    wts_ref[...] = jnp.concatenate([m1, m2], axis=1)
    idx_ref[...] = jnp.concatenate([i1, i2], axis=1)


def _router(x, router_w):
    return pl.pallas_call(
        _router_body,
        out_shape=(jax.ShapeDtypeStruct((T, TOPK), jnp.float32),
                   jax.ShapeDtypeStruct((T, TOPK), jnp.int32)),
    )(x, router_w)


def _dispatch(x, idx, wts):
    idx_flat = idx.reshape(T * TOPK)
    wts_flat = wts.reshape(T * TOPK)
    counts = jnp.bincount(idx_flat, length=E)
    nblk = (counts + TB - 1) // TB
    blk0 = jnp.concatenate([jnp.zeros((1,), jnp.int32),
                            jnp.cumsum(nblk)[:-1].astype(jnp.int32)])
    off = jnp.concatenate([jnp.zeros((1,), jnp.int32),
                           jnp.cumsum(counts)[:-1].astype(jnp.int32)])
    order = jnp.argsort(idx_flat, stable=True)
    sort_e = idx_flat[order]
    rank = jnp.arange(T * TOPK, dtype=jnp.int32) - off[sort_e]
    dest = blk0[sort_e] * TB + rank
    row_token = jnp.zeros((R,), jnp.int32).at[dest].set(
        (order // TOPK).astype(jnp.int32))
    row_scale = jnp.zeros((R,), jnp.float32).at[dest].set(wts_flat[order])
    x_sorted = x[row_token]
    meta = jnp.concatenate([blk0.astype(jnp.int32),
                            nblk.astype(jnp.int32)])
    return x_sorted, row_token, row_scale, meta


def _ffn_body(meta_ref, xs_ref, sc_ref, w1_ref, w2_ref, y_ref, acc_ref):
    e = pl.program_id(0)
    f = pl.program_id(1)
    j = pl.program_id(2)
    nb = meta_ref[E + e]

    @pl.when(j < nb)
    def _():
        x = xs_ref[...]
        h = lax.dot_general(x, w1_ref[0], (((1,), (0,)), ((), ())),
                            preferred_element_type=jnp.float32)
        h = 0.5 * h * (1.0 + lax.erf(h * (2.0 ** -0.5)))
        part = lax.dot_general(h, w2_ref[0], (((1,), (0,)), ((), ())),
                               preferred_element_type=jnp.float32)
        prev = jnp.where(f == 0, 0.0, acc_ref[j])
        acc = part + prev
        acc_ref[j] = acc
        scl = sc_ref[0]
        y = acc.reshape(TB, D // 128, 128) * scl.reshape(TB, 1, 128)
        y_ref[...] = y.reshape(TB, D)


def _xs_map(e, f, j, meta):
    jb = meta[e] + jnp.minimum(j, jnp.maximum(meta[E + e] - 1, 0))
    return (jnp.minimum(jb, NBLK_MAX - 1), 0)


def _sc_map(e, f, j, meta):
    jb = meta[e] + jnp.minimum(j, jnp.maximum(meta[E + e] - 1, 0))
    return (jnp.minimum(jb, NBLK_MAX - 1), 0, 0)


def _ffn(x_sorted, row_scale, w1, w2, meta):
    scale_b = jnp.broadcast_to(row_scale.reshape(NBLK_MAX, TB, 1),
                               (NBLK_MAX, TB, 128))
    grid = (E, FSPLIT, JMAX)
    return pl.pallas_call(
        _ffn_body,
        grid_spec=pltpu.PrefetchScalarGridSpec(
            num_scalar_prefetch=1,
            grid=grid,
            in_specs=[
                pl.BlockSpec((TB, D), _xs_map),
                pl.BlockSpec((1, TB, 128), _sc_map),
                pl.BlockSpec((1, D, FFC), lambda e, f, j, meta: (e, 0, f)),
                pl.BlockSpec((1, FFC, D), lambda e, f, j, meta: (e, f, 0)),
            ],
            out_specs=pl.BlockSpec((TB, D), _xs_map),
            scratch_shapes=[pltpu.VMEM((JMAX, TB, D), jnp.float32)],
        ),
        out_shape=jax.ShapeDtypeStruct((R, D), jnp.float32),
    )(meta, x_sorted, scale_b, w1, w2)


def _combine(y_sorted, row_token, meta):
    total = jnp.sum(meta[E:]) * TB
    valid = jnp.arange(R) < total
    y = jnp.where(valid[:, None], y_sorted, 0.0)
    return jnp.zeros((T, D), jnp.float32).at[row_token].add(y)


def kernel(x, router_w, w1, w2):
    wts, idx = _router(x, router_w)
    x_sorted, row_token, row_scale, meta = _dispatch(x, idx, wts)
    y_sorted = _ffn(x_sorted, row_scale, w1, w2, meta)
    return _combine(y_sorted, row_token, meta)
